# baseline (speedup 1.0000x reference)
import jax
import jax.numpy as jnp
import numpy as np
from jax import lax
from jax.experimental import pallas as pl
from jax.experimental.pallas import tpu as pltpu

NN = 10000
EE = 160000
F_IN_ = 34
HID_ = 256
HEADS_ = 8
DH_ = 32
NB_ = 128
TASK_NAMES = ['logbb', 'logs', 'logp', 'cyp3a4', 'herg', 'ld50']

_BLK = 1000
_NBLK = NN // _BLK
_F32 = jnp.float32


def _gelu(x):
    return jax.nn.gelu(x, approximate=False)


def _gelu_p(x):
    return 0.5 * x * (1.0 + lax.erf(x * np.float32(1.0 / np.sqrt(2.0))))


def _lnx(x, g, b):
    m = jnp.mean(x, axis=-1, keepdims=True)
    v = jnp.mean((x - m) ** 2, axis=-1, keepdims=True)
    return (x - m) / jnp.sqrt(v + 1e-5) * g + b


def _seg_softmax(a, seg, n):
    amax = jax.ops.segment_max(a, seg, num_segments=n)
    amax = jnp.where(jnp.isfinite(amax), amax, 0.0)
    ex = jnp.exp(a - amax[seg])
    den = jax.ops.segment_sum(ex, seg, num_segments=n)
    return ex / (den[seg] + 1e-16)


def _dot(a, b):
    bf = jnp.bfloat16
    return jnp.dot(a.astype(bf), b.astype(bf), preferred_element_type=_F32)


def _mm1_body(h_ref, wt_ref, b_ref, o_ref):
    o_ref[...] = _dot(h_ref[...], wt_ref[...]) + b_ref[...]


def _mm1(h, wt, b):
    kin = h.shape[1]
    return pl.pallas_call(
        _mm1_body,
        grid=(_NBLK,),
        in_specs=[
            pl.BlockSpec((_BLK, kin), lambda i: (i, 0)),
            pl.BlockSpec((kin, HID_), lambda i: (0, 0)),
            pl.BlockSpec((1, HID_), lambda i: (0, 0)),
        ],
        out_specs=pl.BlockSpec((_BLK, HID_), lambda i: (i, 0)),
        out_shape=jax.ShapeDtypeStruct((NN, HID_), _F32),
    )(h, wt, b)


def _mm2_body(h_ref, awt_ref, ab_ref, bwt_ref, bb_ref, oa_ref, ob_ref):
    h = h_ref[...]
    oa_ref[...] = _dot(h, awt_ref[...]) + ab_ref[...]
    ob_ref[...] = _dot(h, bwt_ref[...]) + bb_ref[...]


def _mm2(h, awt, ab, bwt, bb):
    return pl.pallas_call(
        _mm2_body,
        grid=(_NBLK,),
        in_specs=[
            pl.BlockSpec((_BLK, HID_), lambda i: (i, 0)),
            pl.BlockSpec((HID_, HID_), lambda i: (0, 0)),
            pl.BlockSpec((1, HID_), lambda i: (0, 0)),
            pl.BlockSpec((HID_, HID_), lambda i: (0, 0)),
            pl.BlockSpec((1, HID_), lambda i: (0, 0)),
        ],
        out_specs=[
            pl.BlockSpec((_BLK, HID_), lambda i: (i, 0)),
            pl.BlockSpec((_BLK, HID_), lambda i: (i, 0)),
        ],
        out_shape=[
            jax.ShapeDtypeStruct((NN, HID_), _F32),
            jax.ShapeDtypeStruct((NN, HID_), _F32),
        ],
    )(h, awt, ab, bwt, bb)


def _mm3_body(h_ref, awt_ref, ab_ref, bwt_ref, bb_ref, cwt_ref, cb_ref,
              oa_ref, ob_ref, oc_ref):
    h = h_ref[...]
    oa_ref[...] = _dot(h, awt_ref[...]) + ab_ref[...]
    ob_ref[...] = _dot(h, bwt_ref[...]) + bb_ref[...]
    oc_ref[...] = _dot(h, cwt_ref[...]) + cb_ref[...]


def _mm3(h, awt, ab, bwt, bb, cwt, cb):
    return pl.pallas_call(
        _mm3_body,
        grid=(_NBLK,),
        in_specs=[
            pl.BlockSpec((_BLK, HID_), lambda i: (i, 0)),
            pl.BlockSpec((HID_, HID_), lambda i: (0, 0)),
            pl.BlockSpec((1, HID_), lambda i: (0, 0)),
            pl.BlockSpec((HID_, HID_), lambda i: (0, 0)),
            pl.BlockSpec((1, HID_), lambda i: (0, 0)),
            pl.BlockSpec((HID_, HID_), lambda i: (0, 0)),
            pl.BlockSpec((1, HID_), lambda i: (0, 0)),
        ],
        out_specs=[
            pl.BlockSpec((_BLK, HID_), lambda i: (i, 0)),
            pl.BlockSpec((_BLK, HID_), lambda i: (i, 0)),
            pl.BlockSpec((_BLK, HID_), lambda i: (i, 0)),
        ],
        out_shape=[
            jax.ShapeDtypeStruct((NN, HID_), _F32),
            jax.ShapeDtypeStruct((NN, HID_), _F32),
            jax.ShapeDtypeStruct((NN, HID_), _F32),
        ],
    )(h, awt, ab, bwt, bb, cwt, cb)


def _pool_head_body(h_ref, bc_ref, opwt_ref, opb_ref, opg_ref, opbb_ref,
                    s1wt_ref, s1b_ref, s1g_ref, s1bb_ref,
                    s2wt_ref, s2b_ref, s2g_ref, s2bb_ref,
                    w1_ref, b1_ref, w2_ref, b2_ref, w3_ref, b3_ref,
                    o_ref, g_scr):
    def body(b, _):
        mask = bc_ref[...] == b
        hm = jnp.where(mask, h_ref[...], 0.0)
        srow = jnp.sum(hm, axis=0, keepdims=True)
        cnt = jnp.maximum(jnp.sum(mask.astype(_F32)), 1.0)
        hx = jnp.where(mask, h_ref[...], -jnp.inf)
        xrow = jnp.max(hx, axis=0, keepdims=True)
        xrow = jnp.where(jnp.isfinite(xrow), xrow, 0.0)
        g_scr[pl.ds(b, 1), :HID_] = srow / cnt
        g_scr[pl.ds(b, 1), HID_:] = xrow
        return 0
    lax.fori_loop(0, NB_, body, 0)

    def lng(x, g, b):
        m = jnp.mean(x, axis=-1, keepdims=True)
        v = jnp.mean((x - m) ** 2, axis=-1, keepdims=True)
        return _gelu_p((x - m) / jnp.sqrt(v + 1e-5) * g + b)

    g = g_scr[...]
    g = lng(_dot(g, opwt_ref[...]) + opb_ref[...], opg_ref[...], opbb_ref[...])
    s = lng(_dot(g, s1wt_ref[...]) + s1b_ref[...], s1g_ref[...], s1bb_ref[...])
    s = lng(_dot(s, s2wt_ref[...]) + s2b_ref[...], s2g_ref[...], s2bb_ref[...])
    for t in range(6):
        z = _gelu_p(_dot(s, w1_ref[t]) + b1_ref[pl.ds(t, 1)])
        z = _gelu_p(_dot(z, w2_ref[t]) + b2_ref[pl.ds(t, 1)])
        z = _dot(z, w3_ref[t])[:, :1] + b3_ref[pl.ds(t, 1), :1]
        o_ref[:, pl.ds(t, 1)] = z


def _pool_head(hp, bcol, hd_args):
    full = lambda shp: pl.BlockSpec(shp, lambda: tuple(0 for _ in shp))
    in_specs = [
        full((NN + 240, HID_)),
        full((NN + 240, 1)),
        full((2 * HID_, 2 * HID_)), full((1, 2 * HID_)),
        full((1, 2 * HID_)), full((1, 2 * HID_)),
        full((2 * HID_, HID_)), full((1, HID_)), full((1, HID_)), full((1, HID_)),
        full((HID_, HID_)), full((1, HID_)), full((1, HID_)), full((1, HID_)),
        full((6, HID_, 128)), full((6, 128)),
        full((6, 128, 64)), full((6, 64)),
        full((6, 64, 8)), full((6, 8)),
    ]
    return pl.pallas_call(
        _pool_head_body,
        grid=(),
        in_specs=in_specs,
        out_specs=full((NB_, 8)),
        out_shape=jax.ShapeDtypeStruct((NB_, 8), _F32),
        scratch_shapes=[pltpu.VMEM((NB_, 2 * HID_), _F32)],
    )(hp, bcol, *hd_args)


def _edge_gat(xl, xr, att, src_sl, dst_sl):
    xj = xl.reshape(NN, HEADS_, DH_)[src_sl]
    xi = xr.reshape(NN, HEADS_, DH_)[dst_sl]
    e = jax.nn.leaky_relu(xi + xj, negative_slope=0.2)
    a = jnp.sum(e * att, axis=-1)
    a = _seg_softmax(a, dst_sl, NN)
    hn = jax.ops.segment_sum(xj * a[:, :, None], dst_sl, num_segments=NN)
    return hn.reshape(NN, HID_)


def _edge_trans(q, k, v, src, dst):
    q3 = q.reshape(NN, HEADS_, DH_)
    k3 = k.reshape(NN, HEADS_, DH_)
    v3 = v.reshape(NN, HEADS_, DH_)
    a = jnp.sum(q3[dst] * k3[src], axis=-1) / np.sqrt(DH_)
    a = _seg_softmax(a, dst, NN)
    ht = jax.ops.segment_sum(v3[src] * a[:, :, None], dst, num_segments=NN)
    return ht.reshape(NN, HID_)


def kernel(x, params, edge_index, batch):
    f32 = _F32
    src = edge_index[0]
    dst = edge_index[1]
    row = lambda v: v.reshape(1, -1).astype(f32)

    ie = params['ie']
    h = _gelu(_lnx(x @ ie['w'].T + ie['b'], ie['g'], ie['bb']))

    loop = jnp.arange(NN, dtype=src.dtype)
    src_sl = jnp.concatenate([src, loop])
    dst_sl = jnp.concatenate([dst, loop])

    for p in params['gat']:
        xl, xr = _mm2(h, p['lw'].T, row(p['lb']), p['rw'].T, row(p['rb']))
        hn = _edge_gat(xl, xr, p['att'], src_sl, dst_sl) + p['bias']
        h = h + _gelu(_lnx(hn, p['g'], p['bb']))

    t = params['trans']
    q, k, v = _mm3(h, t['qw'].T, row(t['qb']), t['kw'].T, row(t['kb']),
                   t['vw'].T, row(t['vb']))
    ht = _edge_trans(q, k, v, src, dst)
    ht = ht + _mm1(h, t['sw'].T, row(t['sb']))
    h = h + _lnx(ht, t['g'], t['bb'])

    hp = jnp.pad(h, ((0, 240), (0, 0)))
    bcol = jnp.pad(batch, (0, 240), constant_values=NB_).reshape(-1, 1)

    op = params['op']
    hd = params['head']
    tw = hd['towers']
    w1 = jnp.stack([tt['w1'].T for tt in tw])
    b1 = jnp.stack([tt['b1'] for tt in tw])
    w2 = jnp.stack([tt['w2'].T for tt in tw])
    b2 = jnp.stack([tt['b2'] for tt in tw])
    w3 = jnp.stack([jnp.pad(tt['w3'].T, ((0, 0), (0, 7))) for tt in tw])
    b3 = jnp.stack([jnp.pad(tt['b3'], (0, 7)) for tt in tw])
    hd_args = (
        op['w'].T, row(op['b']), row(op['g']), row(op['bb']),
        hd['sw1'].T, row(hd['sb1']), row(hd['sg1']), row(hd['sbb1']),
        hd['sw2'].T, row(hd['sb2']), row(hd['sg2']), row(hd['sbb2']),
        w1, b1, w2, b2, w3, b3,
    )
    out = _pool_head(hp, bcol, hd_args)
    return {name: out[:, i:i + 1] for i, name in enumerate(TASK_NAMES)}
